# baseline (speedup 1.0000x reference)
import functools

import jax
import jax.numpy as jnp
from jax import lax
from jax.experimental import pallas as pl
from jax.experimental.pallas import tpu as pltpu
from jax.experimental.pallas import tpu_sc as plsc

BATCH = 16384
D = 64
NUM_NUMERICAL = 13
N_FIELDS = 2

NC = 2
NS = 16
NW = NC * NS

TOTAL = N_FIELDS * BATCH
B_PER_W = TOTAL // NW
ROWS_PER_W = B_PER_W // 2
CHUNK = 128
N_CHUNKS = B_PER_W // CHUNK
CROWS = CHUNK // 2

_sc_mesh = plsc.VectorSubcoreMesh(core_axis_name="c", subcore_axis_name="s")


@functools.partial(
    pl.kernel,
    out_type=jax.ShapeDtypeStruct((BATCH, 2 * D), jnp.float32),
    mesh=_sc_mesh,
    scratch_types=[
        pltpu.VMEM((B_PER_W,), jnp.int32),
        pltpu.VMEM((CHUNK, 2 * D), jnp.float32),
        pltpu.VMEM((CROWS, 2 * D), jnp.float32),
        pltpu.SemaphoreType.DMA,
    ],
)
def _sc_gather(idx_hbm, t2_hbm, out_hbm, idx_v, wide_v, packed_v, sem):
    wid = lax.axis_index("s") * NC + lax.axis_index("c")
    pltpu.sync_copy(idx_hbm.at[wid], idx_v)

    def chunk_body(g, carry):
        copies = []
        for q in range(CHUNK // 16):
            vec = idx_v[pl.ds(g * CHUNK + q * 16, 16)]
            vq = lax.shift_right_logical(vec, 1)
            for t in range(16):
                i = q * 16 + t
                copies.append(
                    pltpu.async_copy(
                        t2_hbm.at[pl.ds(vq[t], 1)],
                        wide_v.at[pl.ds(i, 1)],
                        sem,
                    )
                )
        for c in copies:
            c.wait()
        for q in range(CHUNK // 16):
            vec = idx_v[pl.ds(g * CHUNK + q * 16, 16)]
            vp = lax.bitwise_and(vec, 1) * D
            for t in range(16):
                i = q * 16 + t
                off = vp[t]
                dst0 = (i % 2) * D
                for c4 in range(D // 16):
                    packed_v[i // 2, pl.ds(dst0 + c4 * 16, 16)] = (
                        wide_v[i, pl.ds(off + c4 * 16, 16)]
                    )
        pltpu.sync_copy(
            packed_v, out_hbm.at[pl.ds(wid * ROWS_PER_W + g * CROWS, CROWS)]
        )
        return carry

    lax.fori_loop(0, N_CHUNKS, chunk_body, 0, unroll=False)


BLK = 2048


def _mlp_body(num_ref, emb_ref, w1n_ref, w1c_ref, b1_ref,
              w2_ref, b2_ref, w3t_ref, b3_ref, out_ref):
    h = (jnp.dot(num_ref[...], w1n_ref[...], preferred_element_type=jnp.float32)
         + jnp.dot(emb_ref[...], w1c_ref[...], preferred_element_type=jnp.float32)
         + b1_ref[...])
    h = jnp.maximum(h, 0.0)
    h2 = jnp.dot(h, w2_ref[...], preferred_element_type=jnp.float32) + b2_ref[...]
    h2 = jnp.maximum(h2, 0.0)
    out_ref[...] = jnp.sum(h2 * w3t_ref[...], axis=1, keepdims=True) + b3_ref[...]


def _mlp(num, emb, w1n, w1c, b1, w2, b2, w3t, b3):
    grid = (BATCH // BLK,)
    full = lambda i: (0, 0)
    row = lambda i: (i, 0)
    return pl.pallas_call(
        _mlp_body,
        grid=grid,
        in_specs=[
            pl.BlockSpec((BLK, NUM_NUMERICAL), row),
            pl.BlockSpec((BLK, 2 * D), row),
            pl.BlockSpec((NUM_NUMERICAL, 128), full),
            pl.BlockSpec((2 * D, 128), full),
            pl.BlockSpec((1, 128), full),
            pl.BlockSpec((128, D), full),
            pl.BlockSpec((1, D), full),
            pl.BlockSpec((1, D), full),
            pl.BlockSpec((1, 1), full),
        ],
        out_specs=pl.BlockSpec((BLK, 1), row),
        out_shape=jax.ShapeDtypeStruct((BATCH, 1), jnp.float32),
    )(num, emb, w1n, w1c, b1, w2, b2, w3t, b3)


def kernel(numerical_features, categorical_features, table, W1, b1, W2, b2, W3, b3):
    idx = categorical_features.astype(jnp.int32).T.reshape(NW, B_PER_W)
    t2 = jnp.concatenate([table[0::2], table[1::2]], axis=1)
    emb = _sc_gather(idx, t2)
    w1n = W1[:NUM_NUMERICAL]
    w1c = W1[NUM_NUMERICAL:]
    return _mlp(numerical_features, emb, w1n, w1c,
                b1.reshape(1, -1), W2, b2.reshape(1, -1),
                W3.reshape(1, -1), b3.reshape(1, 1))


# speedup vs baseline: 22.0315x; 22.0315x over previous
import functools

import jax
import jax.numpy as jnp
from jax import lax
from jax.experimental import pallas as pl
from jax.experimental.pallas import tpu as pltpu
from jax.experimental.pallas import tpu_sc as plsc

BATCH = 16384
D = 64
NUM_NUMERICAL = 13
N_FIELDS = 2

NC = 2
NS = 16
NW = NC * NS

TOTAL = N_FIELDS * BATCH
B_PER_W = TOTAL // NW
ROWS_PER_W = B_PER_W // 2
CHUNK = 128
N_CHUNKS = B_PER_W // CHUNK
CROWS = CHUNK // 2

_sc_mesh = plsc.VectorSubcoreMesh(core_axis_name="c", subcore_axis_name="s")


@functools.partial(
    pl.kernel,
    out_type=jax.ShapeDtypeStruct((BATCH, 2 * D), jnp.float32),
    mesh=_sc_mesh,
    scratch_types=[
        pltpu.VMEM((B_PER_W,), jnp.int32),
        pltpu.VMEM((CROWS, D), jnp.float32),
        pltpu.VMEM((CROWS, D), jnp.float32),
        pltpu.VMEM((CROWS, 2 * D), jnp.float32),
        pltpu.SemaphoreType.DMA,
    ],
)
def _sc_gather(idx_hbm, table_hbm, out_hbm, idx_v, rows_a, rows_b, packed_v, sem):
    wid = lax.axis_index("s") * NC + lax.axis_index("c")
    pltpu.sync_copy(idx_hbm.at[wid], idx_v)

    def chunk_body(g, carry):
        copies = []
        for q in range(CHUNK // 16):
            vec = idx_v[pl.ds(g * CHUNK + q * 16, 16)]
            for t in range(16):
                i = q * 16 + t
                dst = rows_a if i % 2 == 0 else rows_b
                copies.append(
                    pltpu.async_copy(
                        table_hbm.at[pl.ds(vec[t], 1)],
                        dst.at[pl.ds(i // 2, 1)],
                        sem,
                    )
                )
        for c in copies:
            c.wait()
        for k in range(CROWS):
            for c4 in range(D // 16):
                packed_v[k, pl.ds(c4 * 16, 16)] = rows_a[k, pl.ds(c4 * 16, 16)]
                packed_v[k, pl.ds(D + c4 * 16, 16)] = rows_b[k, pl.ds(c4 * 16, 16)]
        pltpu.sync_copy(
            packed_v, out_hbm.at[pl.ds(wid * ROWS_PER_W + g * CROWS, CROWS)]
        )
        return carry

    lax.fori_loop(0, N_CHUNKS, chunk_body, 0, unroll=False)


BLK = 4096


def _mlp_body(num_ref, emb_ref, w1n_ref, w1c_ref, b1_ref,
              w2_ref, b2_ref, w3t_ref, b3_ref, out_ref):
    h = (jnp.dot(num_ref[...], w1n_ref[...], preferred_element_type=jnp.float32)
         + jnp.dot(emb_ref[...], w1c_ref[...], preferred_element_type=jnp.float32)
         + b1_ref[...])
    h = jnp.maximum(h, 0.0)
    h2 = jnp.dot(h, w2_ref[...], preferred_element_type=jnp.float32) + b2_ref[...]
    h2 = jnp.maximum(h2, 0.0)
    out_ref[...] = jnp.sum(h2 * w3t_ref[...], axis=1, keepdims=True) + b3_ref[...]


def _mlp(num, emb, w1n, w1c, b1, w2, b2, w3t, b3):
    grid = (BATCH // BLK,)
    full = lambda i: (0, 0)
    row = lambda i: (i, 0)
    return pl.pallas_call(
        _mlp_body,
        grid=grid,
        in_specs=[
            pl.BlockSpec((BLK, NUM_NUMERICAL), row),
            pl.BlockSpec((BLK, 2 * D), row),
            pl.BlockSpec((NUM_NUMERICAL, 128), full),
            pl.BlockSpec((2 * D, 128), full),
            pl.BlockSpec((1, 128), full),
            pl.BlockSpec((128, D), full),
            pl.BlockSpec((1, D), full),
            pl.BlockSpec((1, D), full),
            pl.BlockSpec((1, 1), full),
        ],
        out_specs=pl.BlockSpec((BLK, 1), row),
        out_shape=jax.ShapeDtypeStruct((BATCH, 1), jnp.float32),
    )(num, emb, w1n, w1c, b1, w2, b2, w3t, b3)


def kernel(numerical_features, categorical_features, table, W1, b1, W2, b2, W3, b3):
    idx = categorical_features.astype(jnp.int32).T.reshape(NW, B_PER_W)
    emb = _sc_gather(idx, table)
    w1n = W1[:NUM_NUMERICAL]
    w1c = W1[NUM_NUMERICAL:]
    return _mlp(numerical_features, emb, w1n, w1c,
                b1.reshape(1, -1), W2, b2.reshape(1, -1),
                W3.reshape(1, -1), b3.reshape(1, 1))
